# baseline (speedup 1.0000x reference)
import jax
import jax.numpy as jnp
from jax.experimental import pallas as pl
from jax.experimental.pallas import tpu as pltpu

NS = 1
SEQ = 2048
H = 768
MLP = 3072
E = 8
K = 2

TILE_T = 128
NPAIR = SEQ * K
NT = NPAIR // TILE_T + E
NPAD = NT * TILE_T
M_TILE = 768
M_STEPS = MLP // M_TILE


def _router_body(x_ref, Wr_ref, br_ref,
                 i1_ref, i2_ref, w1_ref, w2_ref, aux_ref, xb_ref):
    x = x_ref[...]
    logits = jax.lax.dot(x, Wr_ref[...], preferred_element_type=jnp.float32)
    logits = logits + br_ref[...]
    mx = jnp.max(logits, axis=1, keepdims=True)
    ex = jnp.exp(logits - mx)
    probs = ex / jnp.sum(ex, axis=1, keepdims=True)

    lane = jax.lax.broadcasted_iota(jnp.int32, (SEQ, E), 1)
    m1 = jnp.max(probs, axis=1, keepdims=True)
    i1 = jnp.min(jnp.where(probs == m1, lane, E), axis=1, keepdims=True)
    pm = jnp.where(lane == i1, -jnp.inf, probs)
    m2 = jnp.max(pm, axis=1, keepdims=True)
    i2 = jnp.min(jnp.where(pm == m2, lane, E), axis=1, keepdims=True)
    denom = m1 + m2 + 1e-9

    i1_ref[...] = i1
    i2_ref[...] = i2
    w1_ref[...] = m1 / denom
    w2_ref[...] = m2 / denom

    importance = jnp.sum(probs, axis=0)
    load = jnp.sum((probs > 0).astype(jnp.float32), axis=0)
    il = importance * load
    mean = jnp.sum(il) / E
    aux_ref[...] = (jnp.sum((il - mean) ** 2) / E * 0.01).reshape(1, 1)

    xb_ref[...] = x.astype(jnp.bfloat16)


def _router(x, Wr, br):
    return pl.pallas_call(
        _router_body,
        out_shape=[
            jax.ShapeDtypeStruct((SEQ, 1), jnp.int32),
            jax.ShapeDtypeStruct((SEQ, 1), jnp.int32),
            jax.ShapeDtypeStruct((SEQ, 1), jnp.float32),
            jax.ShapeDtypeStruct((SEQ, 1), jnp.float32),
            jax.ShapeDtypeStruct((1, 1), jnp.float32),
            jax.ShapeDtypeStruct((SEQ, H), jnp.bfloat16),
        ],
    )(x, Wr, br.reshape(1, E))


def _expert_body(e_ref, xs_ref, W1_ref, b1_ref, W2_ref, b2_ref, w_ref,
                 out_ref, acc_ref):
    m = pl.program_id(0)
    t = pl.program_id(1)
    rows = pl.ds(t * TILE_T, TILE_T)

    x = xs_ref[rows, :]
    hm = jax.lax.dot(x, W1_ref[0], preferred_element_type=jnp.float32)
    hm = hm + b1_ref[0]
    hm = jax.nn.gelu(hm, approximate=True)
    part = jax.lax.dot(hm.astype(jnp.bfloat16), W2_ref[0],
                       preferred_element_type=jnp.float32)

    @pl.when(m == 0)
    def _init():
        acc_ref[rows, :] = part + b2_ref[0]

    @pl.when(m != 0)
    def _acc():
        acc_ref[rows, :] = acc_ref[rows, :] + part

    @pl.when(m == M_STEPS - 1)
    def _emit():
        out_ref[...] = acc_ref[rows, :] * w_ref[0]


def _expert_mlp(e_of_tile, xs, W1b, b1, W2b, b2, sorted_w):
    grid_spec = pltpu.PrefetchScalarGridSpec(
        num_scalar_prefetch=1,
        grid=(M_STEPS, NT),
        in_specs=[
            pl.BlockSpec((NPAD, H), lambda m, t, e_ref: (0, 0)),
            pl.BlockSpec((1, H, M_TILE), lambda m, t, e_ref: (e_ref[t], 0, m)),
            pl.BlockSpec((1, 1, M_TILE), lambda m, t, e_ref: (e_ref[t], 0, m)),
            pl.BlockSpec((1, M_TILE, H), lambda m, t, e_ref: (e_ref[t], m, 0)),
            pl.BlockSpec((1, 1, H), lambda m, t, e_ref: (e_ref[t], 0, 0)),
            pl.BlockSpec((1, TILE_T, 1), lambda m, t, e_ref: (t, 0, 0)),
        ],
        out_specs=pl.BlockSpec(
            (TILE_T, H),
            lambda m, t, e_ref: (jnp.where(m == M_STEPS - 1, t, NT), 0)),
        scratch_shapes=[pltpu.VMEM((NPAD, H), jnp.float32)],
    )
    return pl.pallas_call(
        _expert_body,
        grid_spec=grid_spec,
        out_shape=jax.ShapeDtypeStruct(((NT + 1) * TILE_T, H), jnp.float32),
    )(e_of_tile, xs, W1b, b1.reshape(E, 1, MLP), W2b, b2.reshape(E, 1, H),
      sorted_w)


def kernel(inputs, Wr, br, W1, b1, W2, b2):
    x = inputs.reshape(SEQ, H)
    W1b = W1.astype(jnp.bfloat16)
    W2b = W2.astype(jnp.bfloat16)

    i1, i2, wa, wb, aux, xb = _router(x, Wr, br)

    e_pair = jnp.concatenate([i1, i2], axis=1).reshape(NPAIR)
    w_pair = jnp.concatenate([wa, wb], axis=1).reshape(NPAIR)
    onehot = (e_pair[:, None] == jnp.arange(E)[None, :]).astype(jnp.int32)
    cum = jnp.cumsum(onehot, axis=0)
    rank = jnp.sum(jnp.where(onehot == 1, cum - 1, 0), axis=1)
    counts = cum[-1]
    tiles_e = (counts + TILE_T - 1) // TILE_T
    bound = jnp.cumsum(tiles_e)
    row_start = jnp.concatenate([jnp.zeros(1, jnp.int32),
                                 bound[:-1].astype(jnp.int32)]) * TILE_T
    pos = row_start[e_pair] + rank
    sorted_tok = jnp.zeros((NPAD,), jnp.int32).at[pos].set(
        jnp.arange(NPAIR, dtype=jnp.int32) // K)
    sorted_w = jnp.zeros((NPAD,), jnp.float32).at[pos].set(w_pair)
    tid = jnp.arange(NT)
    e_of_tile = jnp.minimum(
        jnp.sum((tid[:, None] >= bound[None, :]).astype(jnp.int32), axis=1),
        E - 1).astype(jnp.int32)

    xs = jnp.take(xb, sorted_tok, axis=0)
    ys = _expert_mlp(e_of_tile, xs, W1b, b1, W2b, b2,
                     sorted_w.reshape(NT, TILE_T, 1))

    posr = pos.reshape(SEQ, K)
    out = jnp.take(ys, posr[:, 0], axis=0) + jnp.take(ys, posr[:, 1], axis=0)
    return out.reshape(NS, SEQ, H), aux.reshape(())


# speedup vs baseline: 1.2666x; 1.2666x over previous
import jax
import jax.numpy as jnp
from jax.experimental import pallas as pl
from jax.experimental.pallas import tpu as pltpu

NS = 1
SEQ = 2048
H = 768
MLP = 3072
E = 8
K = 2

TILE_T = 256
NPAIR = SEQ * K
NT = NPAIR // TILE_T + E
NPAD = NT * TILE_T


def _router_body(x_ref, Wr_ref, br_ref,
                 i1_ref, i2_ref, w1_ref, w2_ref, aux_ref, xb_ref):
    x = x_ref[...]
    logits = jax.lax.dot(x, Wr_ref[...], preferred_element_type=jnp.float32)
    logits = logits + br_ref[...]
    mx = jnp.max(logits, axis=1, keepdims=True)
    ex = jnp.exp(logits - mx)
    probs = ex / jnp.sum(ex, axis=1, keepdims=True)

    lane = jax.lax.broadcasted_iota(jnp.int32, (SEQ, E), 1)
    m1 = jnp.max(probs, axis=1, keepdims=True)
    i1 = jnp.min(jnp.where(probs == m1, lane, E), axis=1, keepdims=True)
    pm = jnp.where(lane == i1, -jnp.inf, probs)
    m2 = jnp.max(pm, axis=1, keepdims=True)
    i2 = jnp.min(jnp.where(pm == m2, lane, E), axis=1, keepdims=True)
    denom = m1 + m2 + 1e-9

    i1_ref[...] = i1
    i2_ref[...] = i2
    w1_ref[...] = m1 / denom
    w2_ref[...] = m2 / denom

    importance = jnp.sum(probs, axis=0)
    load = jnp.sum((probs > 0).astype(jnp.float32), axis=0)
    il = importance * load
    mean = jnp.sum(il) / E
    aux_ref[...] = (jnp.sum((il - mean) ** 2) / E * 0.01).reshape(1, 1)

    xb_ref[...] = x.astype(jnp.bfloat16)


def _router(x, Wr, br):
    return pl.pallas_call(
        _router_body,
        out_shape=[
            jax.ShapeDtypeStruct((SEQ, 1), jnp.int32),
            jax.ShapeDtypeStruct((SEQ, 1), jnp.int32),
            jax.ShapeDtypeStruct((SEQ, 1), jnp.float32),
            jax.ShapeDtypeStruct((SEQ, 1), jnp.float32),
            jax.ShapeDtypeStruct((1, 1), jnp.float32),
            jax.ShapeDtypeStruct((SEQ, H), jnp.bfloat16),
        ],
    )(x, Wr, br.reshape(1, E))


def _expert_body(e_ref, xs_ref, W1_ref, b1_ref, W2_ref, b2_ref, w_ref,
                 out_ref):
    x = xs_ref[0]
    hm = jax.lax.dot(x, W1_ref[0], preferred_element_type=jnp.float32)
    hm = hm + b1_ref[0]
    hm = jax.nn.gelu(hm, approximate=True)
    y = jax.lax.dot(hm.astype(jnp.bfloat16), W2_ref[0],
                    preferred_element_type=jnp.float32)
    out_ref[...] = (y + b2_ref[0]) * w_ref[0]


def _expert_mlp(e_of_tile, xs, W1b, b1, W2b, b2, sorted_w):
    grid_spec = pltpu.PrefetchScalarGridSpec(
        num_scalar_prefetch=1,
        grid=(NT,),
        in_specs=[
            pl.BlockSpec((1, TILE_T, H), lambda t, e_ref: (t, 0, 0)),
            pl.BlockSpec((1, H, MLP), lambda t, e_ref: (e_ref[t], 0, 0)),
            pl.BlockSpec((1, 1, MLP), lambda t, e_ref: (e_ref[t], 0, 0)),
            pl.BlockSpec((1, MLP, H), lambda t, e_ref: (e_ref[t], 0, 0)),
            pl.BlockSpec((1, 1, H), lambda t, e_ref: (e_ref[t], 0, 0)),
            pl.BlockSpec((1, TILE_T, 1), lambda t, e_ref: (t, 0, 0)),
        ],
        out_specs=pl.BlockSpec((TILE_T, H), lambda t, e_ref: (t, 0)),
    )
    return pl.pallas_call(
        _expert_body,
        grid_spec=grid_spec,
        out_shape=jax.ShapeDtypeStruct((NPAD, H), jnp.float32),
    )(e_of_tile, xs.reshape(NT, TILE_T, H), W1b, b1.reshape(E, 1, MLP), W2b,
      b2.reshape(E, 1, H), sorted_w)


def kernel(inputs, Wr, br, W1, b1, W2, b2):
    x = inputs.reshape(SEQ, H)
    W1b = W1.astype(jnp.bfloat16)
    W2b = W2.astype(jnp.bfloat16)

    i1, i2, wa, wb, aux, xb = _router(x, Wr, br)

    e_pair = jnp.concatenate([i1, i2], axis=1).reshape(NPAIR)
    w_pair = jnp.concatenate([wa, wb], axis=1).reshape(NPAIR)
    onehot = (e_pair[:, None] == jnp.arange(E)[None, :]).astype(jnp.int32)
    cum = jnp.cumsum(onehot, axis=0)
    rank = jnp.sum(jnp.where(onehot == 1, cum - 1, 0), axis=1)
    counts = cum[-1]
    tiles_e = (counts + TILE_T - 1) // TILE_T
    bound = jnp.cumsum(tiles_e)
    row_start = jnp.concatenate([jnp.zeros(1, jnp.int32),
                                 bound[:-1].astype(jnp.int32)]) * TILE_T
    pos = row_start[e_pair] + rank
    sorted_tok = jnp.zeros((NPAD,), jnp.int32).at[pos].set(
        jnp.arange(NPAIR, dtype=jnp.int32) // K)
    sorted_w = jnp.zeros((NPAD,), jnp.float32).at[pos].set(w_pair)
    tid = jnp.arange(NT)
    e_of_tile = jnp.minimum(
        jnp.sum((tid[:, None] >= bound[None, :]).astype(jnp.int32), axis=1),
        E - 1).astype(jnp.int32)

    xs = jnp.take(xb, sorted_tok, axis=0)
    ys = _expert_mlp(e_of_tile, xs, W1b, b1, W2b, b2,
                     sorted_w.reshape(NT, TILE_T, 1))

    posr = pos.reshape(SEQ, K)
    out = jnp.take(ys, posr[:, 0], axis=0) + jnp.take(ys, posr[:, 1], axis=0)
    return out.reshape(NS, SEQ, H), aux.reshape(())


# speedup vs baseline: 1.5239x; 1.2031x over previous
import jax
import jax.numpy as jnp
from jax.experimental import pallas as pl
from jax.experimental.pallas import tpu as pltpu

NS = 1
SEQ = 2048
H = 768
MLP = 3072
E = 8
K = 2

TILE_T = 256
NPAIR = SEQ * K
NT = NPAIR // TILE_T + E
NPAD = NT * TILE_T
ETILE_PAD = 32


def _router_body(x_ref, Wr_ref, br_ref,
                 p1_ref, p2_ref, w1_ref, w2_ref, et_ref, aux_ref, xb_ref):
    x = x_ref[...]
    logits = jax.lax.dot(x, Wr_ref[...], preferred_element_type=jnp.float32)
    logits = logits + br_ref[...]
    mx = jnp.max(logits, axis=1, keepdims=True)
    ex = jnp.exp(logits - mx)
    probs = ex / jnp.sum(ex, axis=1, keepdims=True)

    lane = jax.lax.broadcasted_iota(jnp.int32, (SEQ, E), 1)
    m1 = jnp.max(probs, axis=1, keepdims=True)
    i1 = jnp.min(jnp.where(probs == m1, lane, E), axis=1, keepdims=True)
    sel1 = lane == i1
    pm = jnp.where(sel1, -jnp.inf, probs)
    m2 = jnp.max(pm, axis=1, keepdims=True)
    i2 = jnp.min(jnp.where(pm == m2, lane, E), axis=1, keepdims=True)
    sel2 = lane == i2
    denom = m1 + m2 + 1e-9
    w1_ref[...] = m1 / denom
    w2_ref[...] = m2 / denom

    oh = jnp.where(sel1 | sel2, 1.0, 0.0)
    r = jax.lax.broadcasted_iota(jnp.int32, (SEQ, SEQ), 0)
    c = jax.lax.broadcasted_iota(jnp.int32, (SEQ, SEQ), 1)
    Ltri = jnp.where(r > c, 1.0, 0.0)
    ranks = jax.lax.dot(Ltri, oh, preferred_element_type=jnp.float32)
    counts = jnp.sum(oh, axis=0).reshape(1, E)
    tiles_e = jnp.floor((counts + (TILE_T - 1)) / TILE_T)
    re = jax.lax.broadcasted_iota(jnp.int32, (E, E), 0)
    ce = jax.lax.broadcasted_iota(jnp.int32, (E, E), 1)
    Utri = jnp.where(re <= ce, 1.0, 0.0)
    bound = jax.lax.dot(tiles_e, Utri,
                        preferred_element_type=jnp.float32)
    row_start = (bound - tiles_e) * TILE_T
    base = row_start + ranks
    p1_ref[...] = jnp.sum(jnp.where(sel1, base, 0.0), axis=1,
                          keepdims=True).astype(jnp.int32)
    p2_ref[...] = jnp.sum(jnp.where(sel2, base, 0.0), axis=1,
                          keepdims=True).astype(jnp.int32)

    tid = jax.lax.broadcasted_iota(
        jnp.int32, (ETILE_PAD, E), 0).astype(jnp.float32)
    et = jnp.sum(jnp.where(tid >= bound, 1.0, 0.0), axis=1, keepdims=True)
    et_ref[...] = jnp.minimum(et, E - 1).astype(jnp.int32)

    importance = jnp.sum(probs, axis=0)
    load = jnp.sum((probs > 0).astype(jnp.float32), axis=0)
    il = importance * load
    mean = jnp.sum(il) / E
    aux_ref[...] = (jnp.sum((il - mean) ** 2) / E * 0.01).reshape(1, 1)

    xb_ref[...] = x.astype(jnp.bfloat16)


def _router(x, Wr, br):
    return pl.pallas_call(
        _router_body,
        out_shape=[
            jax.ShapeDtypeStruct((SEQ, 1), jnp.int32),
            jax.ShapeDtypeStruct((SEQ, 1), jnp.int32),
            jax.ShapeDtypeStruct((SEQ, 1), jnp.float32),
            jax.ShapeDtypeStruct((SEQ, 1), jnp.float32),
            jax.ShapeDtypeStruct((ETILE_PAD, 1), jnp.int32),
            jax.ShapeDtypeStruct((1, 1), jnp.float32),
            jax.ShapeDtypeStruct((SEQ, H), jnp.bfloat16),
        ],
    )(x, Wr, br.reshape(1, E))


def _expert_body(e_ref, xs_ref, W1_ref, b1_ref, W2_ref, b2_ref, w_ref,
                 out_ref):
    x = xs_ref[0]
    w1 = W1_ref[0].astype(jnp.bfloat16)
    hm = jax.lax.dot(x, w1, preferred_element_type=jnp.float32)
    hm = hm + b1_ref[0]
    hm = jax.nn.gelu(hm, approximate=True)
    w2 = W2_ref[0].astype(jnp.bfloat16)
    y = jax.lax.dot(hm.astype(jnp.bfloat16), w2,
                    preferred_element_type=jnp.float32)
    out_ref[...] = (y + b2_ref[0]) * w_ref[0]


def _expert_mlp(e_of_tile, xs, W1, b1, W2, b2, sorted_w):
    grid_spec = pltpu.PrefetchScalarGridSpec(
        num_scalar_prefetch=1,
        grid=(NT,),
        in_specs=[
            pl.BlockSpec((1, TILE_T, H), lambda t, e_ref: (t, 0, 0)),
            pl.BlockSpec((1, H, MLP), lambda t, e_ref: (e_ref[t], 0, 0)),
            pl.BlockSpec((1, 1, MLP), lambda t, e_ref: (e_ref[t], 0, 0)),
            pl.BlockSpec((1, MLP, H), lambda t, e_ref: (e_ref[t], 0, 0)),
            pl.BlockSpec((1, 1, H), lambda t, e_ref: (e_ref[t], 0, 0)),
            pl.BlockSpec((1, TILE_T, 1), lambda t, e_ref: (t, 0, 0)),
        ],
        out_specs=pl.BlockSpec((TILE_T, H), lambda t, e_ref: (t, 0)),
    )
    return pl.pallas_call(
        _expert_body,
        grid_spec=grid_spec,
        out_shape=jax.ShapeDtypeStruct((NPAD, H), jnp.float32),
    )(e_of_tile, xs.reshape(NT, TILE_T, H), W1, b1.reshape(E, 1, MLP), W2,
      b2.reshape(E, 1, H), sorted_w)


def kernel(inputs, Wr, br, W1, b1, W2, b2):
    x = inputs.reshape(SEQ, H)

    p1, p2, wa, wb, et, aux, xb = _router(x, Wr, br)

    pos = jnp.concatenate([p1, p2], axis=1).reshape(NPAIR)
    w_pair = jnp.concatenate([wa, wb], axis=1).reshape(NPAIR)
    sorted_tok = jnp.zeros((NPAD,), jnp.int32).at[pos].set(
        jnp.arange(NPAIR, dtype=jnp.int32) // K)
    sorted_w = jnp.zeros((NPAD,), jnp.float32).at[pos].set(w_pair)
    e_of_tile = et.reshape(ETILE_PAD)[:NT]

    xs = jnp.take(xb, sorted_tok, axis=0)
    ys = _expert_mlp(e_of_tile, xs, W1, b1, W2, b2,
                     sorted_w.reshape(NT, TILE_T, 1))

    out = jnp.take(ys, p1[:, 0], axis=0) + jnp.take(ys, p2[:, 0], axis=0)
    return out.reshape(NS, SEQ, H), aux.reshape(())


# speedup vs baseline: 1.7894x; 1.1742x over previous
import jax
import jax.numpy as jnp
from jax.experimental import pallas as pl
from jax.experimental.pallas import tpu as pltpu

NS = 1
SEQ = 2048
H = 768
MLP = 3072
E = 8
K = 2

TILE_T = 384
NPAIR = SEQ * K
NT = NPAIR // TILE_T + E
NPAD = NT * TILE_T
ETILE_PAD = 24


def _router_body(x_ref, Wr_ref, br_ref,
                 p1_ref, p2_ref, w1_ref, w2_ref, et_ref, nt_ref, aux_ref,
                 xb_ref):
    x = x_ref[...]
    logits = jax.lax.dot(x, Wr_ref[...], preferred_element_type=jnp.float32)
    logits = logits + br_ref[...]
    mx = jnp.max(logits, axis=1, keepdims=True)
    ex = jnp.exp(logits - mx)
    probs = ex / jnp.sum(ex, axis=1, keepdims=True)

    lane = jax.lax.broadcasted_iota(jnp.int32, (SEQ, E), 1)
    m1 = jnp.max(probs, axis=1, keepdims=True)
    i1 = jnp.min(jnp.where(probs == m1, lane, E), axis=1, keepdims=True)
    sel1 = lane == i1
    pm = jnp.where(sel1, -jnp.inf, probs)
    m2 = jnp.max(pm, axis=1, keepdims=True)
    i2 = jnp.min(jnp.where(pm == m2, lane, E), axis=1, keepdims=True)
    sel2 = lane == i2
    denom = m1 + m2 + 1e-9
    w1_ref[...] = m1 / denom
    w2_ref[...] = m2 / denom

    oh = jnp.where(sel1 | sel2, 1.0, 0.0)
    r = jax.lax.broadcasted_iota(jnp.int32, (SEQ, SEQ), 0)
    c = jax.lax.broadcasted_iota(jnp.int32, (SEQ, SEQ), 1)
    Ltri = jnp.where(r > c, 1.0, 0.0)
    ranks = jax.lax.dot(Ltri, oh, preferred_element_type=jnp.float32)
    counts = jnp.sum(oh, axis=0).reshape(1, E)
    tiles_e = jnp.floor((counts + (TILE_T - 1)) / TILE_T)
    re = jax.lax.broadcasted_iota(jnp.int32, (E, E), 0)
    ce = jax.lax.broadcasted_iota(jnp.int32, (E, E), 1)
    Utri = jnp.where(re <= ce, 1.0, 0.0)
    bound = jax.lax.dot(tiles_e, Utri,
                        preferred_element_type=jnp.float32)
    row_start = (bound - tiles_e) * TILE_T
    base = row_start + ranks
    p1_ref[...] = jnp.sum(jnp.where(sel1, base, 0.0), axis=1,
                          keepdims=True).astype(jnp.int32)
    p2_ref[...] = jnp.sum(jnp.where(sel2, base, 0.0), axis=1,
                          keepdims=True).astype(jnp.int32)

    tid = jax.lax.broadcasted_iota(
        jnp.int32, (ETILE_PAD, E), 0).astype(jnp.float32)
    et = jnp.sum(jnp.where(tid >= bound, 1.0, 0.0), axis=1, keepdims=True)
    et_ref[...] = jnp.minimum(et, E - 1).astype(jnp.int32)
    nt_ref[...] = jnp.max(bound).astype(jnp.int32).reshape(1, 1)

    importance = jnp.sum(probs, axis=0)
    load = jnp.sum((probs > 0).astype(jnp.float32), axis=0)
    il = importance * load
    mean = jnp.sum(il) / E
    aux_ref[...] = (jnp.sum((il - mean) ** 2) / E * 0.01).reshape(1, 1)

    xb_ref[...] = x.astype(jnp.bfloat16)


def _router(x, Wr, br):
    return pl.pallas_call(
        _router_body,
        out_shape=[
            jax.ShapeDtypeStruct((SEQ, 1), jnp.int32),
            jax.ShapeDtypeStruct((SEQ, 1), jnp.int32),
            jax.ShapeDtypeStruct((SEQ, 1), jnp.float32),
            jax.ShapeDtypeStruct((SEQ, 1), jnp.float32),
            jax.ShapeDtypeStruct((ETILE_PAD, 1), jnp.int32),
            jax.ShapeDtypeStruct((1, 1), jnp.int32),
            jax.ShapeDtypeStruct((1, 1), jnp.float32),
            jax.ShapeDtypeStruct((SEQ, H), jnp.bfloat16),
        ],
    )(x, Wr, br.reshape(1, E))


def _expert_body(e_ref, n_ref, xs_ref, W1_ref, b1_ref, W2_ref, b2_ref,
                 out_ref):
    t = pl.program_id(0)

    @pl.when(t < n_ref[0])
    def _compute():
        x = xs_ref[0]
        w1 = W1_ref[0].astype(jnp.bfloat16)
        hm = jax.lax.dot(x, w1, preferred_element_type=jnp.float32)
        hm = hm + b1_ref[0]
        hm = jax.nn.gelu(hm, approximate=True)
        w2 = W2_ref[0].astype(jnp.bfloat16)
        out_ref[...] = jax.lax.dot(
            hm.astype(jnp.bfloat16), w2,
            preferred_element_type=jnp.float32) + b2_ref[0]


def _expert_mlp(e_of_tile, nt_real, xs, W1, b1, W2, b2):
    grid_spec = pltpu.PrefetchScalarGridSpec(
        num_scalar_prefetch=2,
        grid=(NT,),
        in_specs=[
            pl.BlockSpec((1, TILE_T, H), lambda t, e_ref, n_ref: (t, 0, 0)),
            pl.BlockSpec((1, H, MLP),
                         lambda t, e_ref, n_ref: (e_ref[t], 0, 0)),
            pl.BlockSpec((1, 1, MLP),
                         lambda t, e_ref, n_ref: (e_ref[t], 0, 0)),
            pl.BlockSpec((1, MLP, H),
                         lambda t, e_ref, n_ref: (e_ref[t], 0, 0)),
            pl.BlockSpec((1, 1, H),
                         lambda t, e_ref, n_ref: (e_ref[t], 0, 0)),
        ],
        out_specs=pl.BlockSpec((TILE_T, H), lambda t, e_ref, n_ref: (t, 0)),
    )
    return pl.pallas_call(
        _expert_body,
        grid_spec=grid_spec,
        out_shape=jax.ShapeDtypeStruct((NPAD, H), jnp.float32),
    )(e_of_tile, nt_real, xs.reshape(NT, TILE_T, H), W1,
      b1.reshape(E, 1, MLP), W2, b2.reshape(E, 1, H))


def kernel(inputs, Wr, br, W1, b1, W2, b2):
    x = inputs.reshape(SEQ, H)

    p1, p2, wa, wb, et, ntr, aux, xb = _router(x, Wr, br)

    sorted_tok = (jnp.zeros((NPAD,), jnp.int32)
                  .at[p1[:, 0]].set(jnp.arange(SEQ, dtype=jnp.int32))
                  .at[p2[:, 0]].set(jnp.arange(SEQ, dtype=jnp.int32)))
    e_of_tile = et.reshape(ETILE_PAD)[:NT]
    nt_real = ntr.reshape(1)

    xs = jnp.take(xb, sorted_tok, axis=0)
    ys = _expert_mlp(e_of_tile, nt_real, xs, W1, b1, W2, b2)

    out = (jnp.take(ys, p1[:, 0], axis=0) * wa
           + jnp.take(ys, p2[:, 0], axis=0) * wb)
    return out.reshape(NS, SEQ, H), aux.reshape(())


# speedup vs baseline: 2.2429x; 1.2534x over previous
import functools

import jax
import jax.numpy as jnp
from jax import lax
from jax.experimental import pallas as pl
from jax.experimental.pallas import tpu as pltpu
from jax.experimental.pallas import tpu_sc as plsc

NS = 1
SEQ = 2048
H = 768
MLP = 3072
E = 8
K = 2

TILE_T = 384
NPAIR = SEQ * K
NT = NPAIR // TILE_T + E
NPAD = NT * TILE_T
ETILE_PAD = 24

NW = 32
TOK_W = SEQ // NW


def _router_body(x_ref, Wr_ref, br_ref,
                 p1_ref, p2_ref, w1_ref, w2_ref, et_ref, nt_ref, aux_ref):
    x = x_ref[...]
    logits = jax.lax.dot(x, Wr_ref[...], preferred_element_type=jnp.float32)
    logits = logits + br_ref[...]
    mx = jnp.max(logits, axis=1, keepdims=True)
    ex = jnp.exp(logits - mx)
    probs = ex / jnp.sum(ex, axis=1, keepdims=True)

    lane = jax.lax.broadcasted_iota(jnp.int32, (SEQ, E), 1)
    m1 = jnp.max(probs, axis=1, keepdims=True)
    i1 = jnp.min(jnp.where(probs == m1, lane, E), axis=1, keepdims=True)
    sel1 = lane == i1
    pm = jnp.where(sel1, -jnp.inf, probs)
    m2 = jnp.max(pm, axis=1, keepdims=True)
    i2 = jnp.min(jnp.where(pm == m2, lane, E), axis=1, keepdims=True)
    sel2 = lane == i2
    denom = m1 + m2 + 1e-9
    w1_ref[...] = m1 / denom
    w2_ref[...] = m2 / denom

    oh = jnp.where(sel1 | sel2, 1.0, 0.0)
    r = jax.lax.broadcasted_iota(jnp.int32, (SEQ, SEQ), 0)
    c = jax.lax.broadcasted_iota(jnp.int32, (SEQ, SEQ), 1)
    Ltri = jnp.where(r > c, 1.0, 0.0)
    ranks = jax.lax.dot(Ltri, oh, preferred_element_type=jnp.float32)
    counts = jnp.sum(oh, axis=0).reshape(1, E)
    tiles_e = jnp.floor((counts + (TILE_T - 1)) / TILE_T)
    re = jax.lax.broadcasted_iota(jnp.int32, (E, E), 0)
    ce = jax.lax.broadcasted_iota(jnp.int32, (E, E), 1)
    Utri = jnp.where(re <= ce, 1.0, 0.0)
    bound = jax.lax.dot(tiles_e, Utri,
                        preferred_element_type=jnp.float32)
    row_start = (bound - tiles_e) * TILE_T
    base = row_start + ranks
    p1_ref[...] = jnp.sum(jnp.where(sel1, base, 0.0), axis=1,
                          keepdims=True).astype(jnp.int32)
    p2_ref[...] = jnp.sum(jnp.where(sel2, base, 0.0), axis=1,
                          keepdims=True).astype(jnp.int32)

    tid = jax.lax.broadcasted_iota(
        jnp.int32, (ETILE_PAD, E), 0).astype(jnp.float32)
    et = jnp.sum(jnp.where(tid >= bound, 1.0, 0.0), axis=1, keepdims=True)
    et_ref[...] = jnp.minimum(et, E - 1).astype(jnp.int32)
    nt_ref[...] = jnp.max(bound).astype(jnp.int32).reshape(1, 1)

    importance = jnp.sum(probs, axis=0)
    load = jnp.sum((probs > 0).astype(jnp.float32), axis=0)
    il = importance * load
    mean = jnp.sum(il) / E
    aux_ref[...] = (jnp.sum((il - mean) ** 2) / E * 0.01).reshape(1, 1)


def _router(x, Wr, br):
    return pl.pallas_call(
        _router_body,
        out_shape=[
            jax.ShapeDtypeStruct((SEQ, 1), jnp.int32),
            jax.ShapeDtypeStruct((SEQ, 1), jnp.int32),
            jax.ShapeDtypeStruct((SEQ, 1), jnp.float32),
            jax.ShapeDtypeStruct((SEQ, 1), jnp.float32),
            jax.ShapeDtypeStruct((ETILE_PAD, 1), jnp.int32),
            jax.ShapeDtypeStruct((1, 1), jnp.int32),
            jax.ShapeDtypeStruct((1, 1), jnp.float32),
        ],
    )(x, Wr, br.reshape(1, E))


def _sc_dispatch(x, pidx):
    @functools.partial(
        pl.kernel,
        mesh=plsc.VectorSubcoreMesh(core_axis_name="c", subcore_axis_name="s"),
        out_type=jax.ShapeDtypeStruct((NPAD, H), jnp.float32),
        scratch_types=[
            pltpu.VMEM((TOK_W, H), jnp.float32),
            pltpu.VMEM((TOK_W,), jnp.int32),
            pltpu.VMEM((TOK_W,), jnp.int32),
            pltpu.SemaphoreType.DMA,
            pltpu.SemaphoreType.DMA,
        ],
    )
    def k(x_hbm, pidx_hbm, xs_hbm, rows_v, i1_v, i2_v, s1, s2):
        wid = lax.axis_index("s") * 2 + lax.axis_index("c")
        base = wid * TOK_W
        pltpu.sync_copy(x_hbm.at[pl.ds(base, TOK_W)], rows_v)
        pltpu.sync_copy(pidx_hbm.at[wid, 0], i1_v)
        pltpu.sync_copy(pidx_hbm.at[wid, 1], i2_v)
        c1 = pltpu.async_copy(rows_v, xs_hbm.at[i1_v], s1)
        c2 = pltpu.async_copy(rows_v, xs_hbm.at[i2_v], s2)
        c1.wait()
        c2.wait()

    return k(x, pidx)


def _expert_body(e_ref, n_ref, xs_ref, W1_ref, b1_ref, W2_ref, b2_ref,
                 out_ref):
    t = pl.program_id(0)

    @pl.when(t < n_ref[0])
    def _compute():
        x = xs_ref[0].astype(jnp.bfloat16)
        w1 = W1_ref[0].astype(jnp.bfloat16)
        hm = jax.lax.dot(x, w1, preferred_element_type=jnp.float32)
        hm = hm + b1_ref[0]
        hm = jax.nn.gelu(hm, approximate=True)
        w2 = W2_ref[0].astype(jnp.bfloat16)
        out_ref[...] = jax.lax.dot(
            hm.astype(jnp.bfloat16), w2,
            preferred_element_type=jnp.float32) + b2_ref[0]


def _expert_mlp(e_of_tile, nt_real, xs, W1, b1, W2, b2):
    grid_spec = pltpu.PrefetchScalarGridSpec(
        num_scalar_prefetch=2,
        grid=(NT,),
        in_specs=[
            pl.BlockSpec((1, TILE_T, H), lambda t, e_ref, n_ref: (t, 0, 0)),
            pl.BlockSpec((1, H, MLP),
                         lambda t, e_ref, n_ref: (e_ref[t], 0, 0)),
            pl.BlockSpec((1, 1, MLP),
                         lambda t, e_ref, n_ref: (e_ref[t], 0, 0)),
            pl.BlockSpec((1, MLP, H),
                         lambda t, e_ref, n_ref: (e_ref[t], 0, 0)),
            pl.BlockSpec((1, 1, H),
                         lambda t, e_ref, n_ref: (e_ref[t], 0, 0)),
        ],
        out_specs=pl.BlockSpec((TILE_T, H), lambda t, e_ref, n_ref: (t, 0)),
    )
    return pl.pallas_call(
        _expert_body,
        grid_spec=grid_spec,
        out_shape=jax.ShapeDtypeStruct((NPAD, H), jnp.float32),
    )(e_of_tile, nt_real, xs.reshape(NT, TILE_T, H), W1,
      b1.reshape(E, 1, MLP), W2, b2.reshape(E, 1, H))


def kernel(inputs, Wr, br, W1, b1, W2, b2):
    x = inputs.reshape(SEQ, H)

    p1, p2, wa, wb, et, ntr, aux = _router(x, Wr, br)

    pidx = jnp.stack([p1.reshape(NW, TOK_W), p2.reshape(NW, TOK_W)], axis=1)
    e_of_tile = et.reshape(ETILE_PAD)[:NT]
    nt_real = ntr.reshape(1)

    xs = _sc_dispatch(x, pidx)
    ys = _expert_mlp(e_of_tile, nt_real, xs, W1, b1, W2, b2)

    out = (jnp.take(ys, p1[:, 0], axis=0) * wa
           + jnp.take(ys, p2[:, 0], axis=0) * wb)
    return out.reshape(NS, SEQ, H), aux.reshape(())
